# baseline (speedup 1.0000x reference)
import functools
from itertools import product

import jax
import jax.numpy as jnp
from jax.experimental import pallas as pl
from jax.experimental.pallas import tpu as pltpu

B, L, W = 128, 8, 32
D = W * W
E = 8
H = 2 * D
HW = W * W
PAD = 33


def _col_mask(dw, shape, axis):
    wcol = jax.lax.broadcasted_iota(jnp.int32, shape, axis) % W
    if dw == -1:
        return wcol != 0
    if dw == 1:
        return wcol != (W - 1)
    return None


def _conv_kernel(xp_ref, w1_ref, b1_ref, w2_ref, b2_ref, out_ref, *, nb):
    xb = xp_ref[...]
    taps = []
    for kd, kh, kw in product(range(3), range(3), range(3)):
        s = (kh - 1) * W + (kw - 1)
        sl = xb[:, kd:kd + L, PAD + s:PAD + s + HW]
        m = _col_mask(kw - 1, sl.shape, 2)
        if m is not None:
            sl = jnp.where(m, sl, 0.0)
        taps.append(sl)
    tap = jnp.stack(taps, axis=0).reshape(27, nb * L * HW)
    r1 = jax.lax.dot_general(w1_ref[...], tap, (((1,), (0,)), ((), ())),
                             preferred_element_type=jnp.float32)
    r1 = jnp.maximum(r1 + b1_ref[...], 0.0)
    r1 = r1.reshape(10, nb, L, HW).transpose(1, 0, 2, 3).reshape(nb, 80, HW)
    r1 = jnp.pad(r1, ((0, 0), (0, 0), (PAD, PAD)))
    acc = jnp.zeros((nb, HW), jnp.float32)
    for t, (kh, kw) in enumerate(product(range(3), range(3))):
        s = (kh - 1) * W + (kw - 1)
        sl = r1[:, :, PAD + s:PAD + s + HW]
        m = _col_mask(kw - 1, sl.shape, 2)
        if m is not None:
            sl = jnp.where(m, sl, 0.0)
        acc = acc + jnp.sum(sl * w2_ref[t][None, :, None], axis=1)
    out_ref[...] = jnp.maximum(acc + b2_ref[0, 0], 0.0)


def _erf(x):
    a1, a2, a3 = 0.254829592, -0.284496736, 1.421413741
    a4, a5, p = -1.453152027, 1.061405429, 0.3275911
    s = jnp.sign(x)
    z = jnp.abs(x)
    t = 1.0 / (1.0 + p * z)
    poly = t * (a1 + t * (a2 + t * (a3 + t * (a4 + t * a5))))
    return s * (1.0 - poly * jnp.exp(-z * z))


def _ndtr(z):
    return 0.5 * (1.0 + _erf(z * 0.70710678118654752))


def _cv_sq(v, n):
    mean = jnp.sum(v) / n
    var = jnp.sum((v - mean) ** 2) / (n - 1)
    return var / (mean * mean + 1e-10)


def _gate_kernel(x_ref, wg_ref, wn_ref, nz_ref, g_ref, aux_ref):
    x = x_ref[...]
    aux = jnp.zeros((), jnp.float32)
    for m in range(2):
        clean = jax.lax.dot_general(x, wg_ref[m], (((1,), (0,)), ((), ())),
                                    preferred_element_type=jnp.float32)
        sp = jax.lax.dot_general(x, wn_ref[m], (((1,), (0,)), ((), ())),
                                 preferred_element_type=jnp.float32)
        stddev = jnp.maximum(sp, 0.0) + jnp.log1p(jnp.exp(-jnp.abs(sp))) + 1e-2
        noisy = clean + nz_ref[m] * stddev
        m1 = jnp.max(noisy, axis=1, keepdims=True)
        n2 = jnp.where(noisy >= m1, -jnp.inf, noisy)
        m2 = jnp.max(n2, axis=1, keepdims=True)
        n3 = jnp.where(n2 >= m2, -jnp.inf, n2)
        m3 = jnp.max(n3, axis=1, keepdims=True)
        g1 = 1.0 / (1.0 + jnp.exp(m2 - m1))
        g2 = 1.0 - g1
        gates = jnp.where(noisy == m1, g1,
                          jnp.where(noisy == m2, g2, 0.0))
        g_ref[m] = gates
        is_in = noisy > m3
        p_in = _ndtr((clean - m3) / stddev)
        p_out = _ndtr((clean - m2) / stddev)
        load = jnp.sum(jnp.where(is_in, p_in, p_out), axis=0)
        imp = jnp.sum(gates, axis=0)
        aux = aux + (_cv_sq(imp, E) + _cv_sq(load, E)) * 1e-2
    aux_ref[...] = jnp.reshape(aux, (1, 1))


def _moe_kernel(x_ref, gt_ref, w1_ref, b1_ref, w2_ref, b2_ref, y_ref):
    e = pl.program_id(1)
    x = x_ref[...]
    h = jax.lax.dot_general(x, w1_ref[0, 0], (((1,), (1,)), ((), ())),
                            preferred_element_type=jnp.float32)
    h = jnp.maximum(h + b1_ref[0, 0], 0.0)
    o = jax.lax.dot_general(h, w2_ref[0, 0], (((1,), (1,)), ((), ())),
                            preferred_element_type=jnp.float32)
    o = o + b2_ref[0, 0]
    o = o - jnp.max(o, axis=1, keepdims=True)
    o = jnp.exp(o)
    o = o / jnp.sum(o, axis=1, keepdims=True)
    contrib = gt_ref[0, 0, 0][:, None] * o

    @pl.when(e == 0)
    def _():
        y_ref[0] = contrib

    @pl.when(e != 0)
    def _():
        y_ref[0] = y_ref[0] + contrib


def _combine_kernel(in_ref, y_ref, out_ref):
    yt = y_ref[0][:, None, :]
    ya = y_ref[1][:, None, :]
    out_ref[...] = in_ref[...] * yt + ya


def kernel(input, conv1_w, conv1_b, conv2_w, conv2_b,
           t_w_gate, t_w_noise, t_fc1w, t_fc1b, t_fc2w, t_fc2b,
           a_w_gate, a_w_noise, a_fc1w, a_fc1b, a_fc2w, a_fc2b):
    f32 = jnp.float32
    x0 = input[:, :, 0].reshape(B, L, HW)
    xp = jnp.pad(x0, ((0, 0), (1, 1), (PAD, PAD)))
    w1r = conv1_w[:, 0].reshape(10, 27)
    b1r = conv1_b.reshape(10, 1)
    w2r = conv2_w[0].transpose(2, 3, 0, 1).reshape(9, 80)

    NB = 8
    flat = pl.pallas_call(
        functools.partial(_conv_kernel, nb=NB),
        grid=(B // NB,),
        in_specs=[
            pl.BlockSpec((NB, 10, 2 * PAD + HW), lambda i: (i, 0, 0)),
            pl.BlockSpec((10, 27), lambda i: (0, 0)),
            pl.BlockSpec((10, 1), lambda i: (0, 0)),
            pl.BlockSpec((9, 80), lambda i: (0, 0)),
            pl.BlockSpec((1, 1), lambda i: (0, 0)),
        ],
        out_specs=pl.BlockSpec((NB, HW), lambda i: (i, 0)),
        out_shape=jax.ShapeDtypeStruct((B, HW), f32),
    )(xp, w1r, b1r, w2r, conv2_b.reshape(1, 1))

    noise_t = jax.random.normal(jax.random.key(42), (B, E), f32)
    noise_a = jax.random.normal(jax.random.key(43), (B, E), f32)
    wg = jnp.stack([t_w_gate, a_w_gate])
    wn = jnp.stack([t_w_noise, a_w_noise])
    nz = jnp.stack([noise_t, noise_a])
    fc1w = jnp.stack([t_fc1w, a_fc1w])
    fc1b = jnp.stack([t_fc1b, a_fc1b]).reshape(2, E, 1, H)
    fc2w = jnp.stack([t_fc2w, a_fc2w])
    fc2b = jnp.stack([t_fc2b, a_fc2b]).reshape(2, E, 1, D)

    gates, aux = pl.pallas_call(
        _gate_kernel,
        in_specs=[
            pl.BlockSpec((B, D), lambda: (0, 0)),
            pl.BlockSpec((2, D, E), lambda: (0, 0, 0)),
            pl.BlockSpec((2, D, E), lambda: (0, 0, 0)),
            pl.BlockSpec((2, B, E), lambda: (0, 0, 0)),
        ],
        out_specs=[
            pl.BlockSpec((2, B, E), lambda: (0, 0, 0)),
            pl.BlockSpec((1, 1), lambda: (0, 0)),
        ],
        out_shape=[
            jax.ShapeDtypeStruct((2, B, E), f32),
            jax.ShapeDtypeStruct((1, 1), f32),
        ],
    )(flat, wg, wn, nz)

    gates_t = gates.transpose(0, 2, 1).reshape(2, E, 1, B)

    y = pl.pallas_call(
        _moe_kernel,
        grid=(2, E),
        in_specs=[
            pl.BlockSpec((B, D), lambda m, e: (0, 0)),
            pl.BlockSpec((1, 1, 1, B), lambda m, e: (m, e, 0, 0)),
            pl.BlockSpec((1, 1, H, D), lambda m, e: (m, e, 0, 0)),
            pl.BlockSpec((1, 1, 1, H), lambda m, e: (m, e, 0, 0)),
            pl.BlockSpec((1, 1, D, H), lambda m, e: (m, e, 0, 0)),
            pl.BlockSpec((1, 1, 1, D), lambda m, e: (m, e, 0, 0)),
        ],
        out_specs=pl.BlockSpec((1, B, D), lambda m, e: (m, 0, 0)),
        out_shape=jax.ShapeDtypeStruct((2, B, D), f32),
        compiler_params=pltpu.CompilerParams(
            dimension_semantics=("parallel", "arbitrary")),
    )(flat, gates_t, fc1w, fc1b, fc2w, fc2b)

    NB2 = 16
    out = pl.pallas_call(
        _combine_kernel,
        grid=(B // NB2,),
        in_specs=[
            pl.BlockSpec((NB2, L, HW), lambda i: (i, 0, 0)),
            pl.BlockSpec((2, NB2, HW), lambda i: (0, i, 0)),
        ],
        out_specs=pl.BlockSpec((NB2, L, HW), lambda i: (i, 0, 0)),
        out_shape=jax.ShapeDtypeStruct((B, L, HW), f32),
    )(x0, y)

    return out.reshape(B, L, 1, W, W), aux[0, 0]


# speedup vs baseline: 2.7176x; 2.7176x over previous
import functools
from itertools import product

import jax
import jax.numpy as jnp
from jax.experimental import pallas as pl
from jax.experimental.pallas import tpu as pltpu

B, L, W = 128, 8, 32
D = W * W
E = 8
H = 2 * D
HW = W * W
PAD = 33
DP = 16


def _shift_mask(dh, dw, shape, axis):
    hw = jax.lax.broadcasted_iota(jnp.int32, shape, axis)
    h = hw // W
    w = hw % W
    ok = None
    if dh == -1:
        ok = h >= 1
    elif dh == 1:
        ok = h <= W - 2
    if dw == -1:
        okw = w >= 1
        ok = okw if ok is None else jnp.logical_and(ok, okw)
    elif dw == 1:
        okw = w <= W - 2
        ok = okw if ok is None else jnp.logical_and(ok, okw)
    return ok


def _conv_kernel(xp_ref, m1_ref, b1_ref, w2_ref, b2_ref, out_ref, *, nb):
    xb = xp_ref[...]
    taps = []
    for t, (dh, dw) in enumerate(product((-1, 0, 1), (-1, 0, 1))):
        s = dh * W + dw
        sl = xb[:, :, PAD + s:PAD + s + HW]
        m = _shift_mask(0, dw, (1, 1, HW), 2)
        if m is not None:
            sl = jnp.where(m, sl, 0.0)
        taps.append(sl)
    tap = jnp.stack(taps, axis=1).reshape(nb, 9 * DP, HW)
    m1 = m1_ref[...]
    w2 = w2_ref[...]
    outs = []
    for b in range(nb):
        r1 = jax.lax.dot_general(m1, tap[b], (((1,), (0,)), ((), ())),
                                 preferred_element_type=jnp.float32)
        r1 = jnp.maximum(r1 + b1_ref[...], 0.0)
        z = jax.lax.dot_general(w2, r1, (((1,), (0,)), ((), ())),
                                preferred_element_type=jnp.float32)
        outs.append(z)
    zs = jnp.stack(outs, axis=0)
    zp = jnp.pad(zs, ((0, 0), (0, 0), (PAD, PAD)))
    acc = jnp.zeros((nb, HW), jnp.float32)
    for t, (dh, dw) in enumerate(product((-1, 0, 1), (-1, 0, 1))):
        s = dh * W + dw
        sl = zp[:, t, PAD + s:PAD + s + HW]
        m = _shift_mask(0, dw, (1, HW), 1)
        if m is not None:
            sl = jnp.where(m, sl, 0.0)
        acc = acc + sl
    out_ref[...] = jnp.maximum(acc + b2_ref[0, 0], 0.0)


def _erf(x):
    a1, a2, a3 = 0.254829592, -0.284496736, 1.421413741
    a4, a5, p = -1.453152027, 1.061405429, 0.3275911
    s = jnp.sign(x)
    z = jnp.abs(x)
    t = 1.0 / (1.0 + p * z)
    poly = t * (a1 + t * (a2 + t * (a3 + t * (a4 + t * a5))))
    return s * (1.0 - poly * jnp.exp(-z * z))


def _ndtr(z):
    return 0.5 * (1.0 + _erf(z * 0.70710678118654752))


def _cv_sq(v, n):
    mean = jnp.sum(v) / n
    var = jnp.sum((v - mean) ** 2) / (n - 1)
    return var / (mean * mean + 1e-10)


def _gate_kernel(x_ref, wg_ref, wn_ref, nz_ref, g_ref, aux_ref):
    x = x_ref[...]
    aux = jnp.zeros((), jnp.float32)
    for m in range(2):
        clean = jax.lax.dot_general(x, wg_ref[m], (((1,), (0,)), ((), ())),
                                    preferred_element_type=jnp.float32)
        sp = jax.lax.dot_general(x, wn_ref[m], (((1,), (0,)), ((), ())),
                                 preferred_element_type=jnp.float32)
        stddev = jnp.maximum(sp, 0.0) + jnp.log1p(jnp.exp(-jnp.abs(sp))) + 1e-2
        noisy = clean + nz_ref[m] * stddev
        m1 = jnp.max(noisy, axis=1, keepdims=True)
        n2 = jnp.where(noisy >= m1, -jnp.inf, noisy)
        m2 = jnp.max(n2, axis=1, keepdims=True)
        n3 = jnp.where(n2 >= m2, -jnp.inf, n2)
        m3 = jnp.max(n3, axis=1, keepdims=True)
        g1 = 1.0 / (1.0 + jnp.exp(m2 - m1))
        g2 = 1.0 - g1
        gates = jnp.where(noisy == m1, g1,
                          jnp.where(noisy == m2, g2, 0.0))
        g_ref[m] = gates
        is_in = noisy > m3
        p_in = _ndtr((clean - m3) / stddev)
        p_out = _ndtr((clean - m2) / stddev)
        load = jnp.sum(jnp.where(is_in, p_in, p_out), axis=0)
        imp = jnp.sum(gates, axis=0)
        aux = aux + (_cv_sq(imp, E) + _cv_sq(load, E)) * 1e-2
    aux_ref[...] = jnp.reshape(aux, (1, 1))


def _moe_kernel(x_ref, gt_ref, ga_ref,
                tw1_ref, tb1_ref, tw2_ref, tb2_ref,
                aw1_ref, ab1_ref, aw2_ref, ab2_ref,
                yt_ref, ya_ref, ot_acc, oa_acc):
    e = pl.program_id(0)
    ph = pl.program_id(1)
    x = x_ref[...]

    def half(w1_ref, b1_ref, w2_ref, o_acc):
        h = jax.lax.dot_general(x, w1_ref[0], (((1,), (1,)), ((), ())),
                                preferred_element_type=jnp.float32)
        h = jnp.maximum(h + b1_ref[0, 0], 0.0)
        o = jax.lax.dot_general(h, w2_ref[0], (((1,), (1,)), ((), ())),
                                preferred_element_type=jnp.float32)

        @pl.when(ph == 0)
        def _():
            o_acc[...] = o

        @pl.when(ph == 1)
        def _():
            o_acc[...] = o_acc[...] + o

    def emit(b2_ref, g_ref, o_acc, y_ref):
        o = o_acc[...] + b2_ref[0]
        o = o - jnp.max(o, axis=1, keepdims=True)
        o = jnp.exp(o)
        o = o / jnp.sum(o, axis=1, keepdims=True)
        contrib = g_ref[0, 0, 0][:, None] * o

        @pl.when(e == 0)
        def _():
            y_ref[...] = contrib

        @pl.when(e != 0)
        def _():
            y_ref[...] = y_ref[...] + contrib

    half(tw1_ref, tb1_ref, tw2_ref, ot_acc)
    half(aw1_ref, ab1_ref, aw2_ref, oa_acc)

    @pl.when(ph == 1)
    def _():
        emit(tb2_ref, gt_ref, ot_acc, yt_ref)
        emit(ab2_ref, ga_ref, oa_acc, ya_ref)


def _combine_kernel(in_ref, yt_ref, ya_ref, out_ref):
    out_ref[...] = (in_ref[...] * yt_ref[...][:, None, :]
                    + ya_ref[...][:, None, :])


def kernel(input, conv1_w, conv1_b, conv2_w, conv2_b,
           t_w_gate, t_w_noise, t_fc1w, t_fc1b, t_fc2w, t_fc2b,
           a_w_gate, a_w_noise, a_fc1w, a_fc1b, a_fc2w, a_fc2b):
    f32 = jnp.float32
    x0 = input[:, :, 0].reshape(B, L, HW)
    xp = jnp.pad(x0, ((0, 0), (1, DP - 1 - L), (PAD, PAD)))
    w1f = conv1_w[:, 0].reshape(10, 3, 9)
    kd_i = jnp.arange(3)[:, None, None]
    d_i = jnp.arange(L)[None, :, None]
    dd_i = jnp.arange(DP)[None, None, :]
    onehot = (dd_i == d_i + kd_i).astype(f32)
    m1 = jnp.einsum('ckt,kdj->cdtj', w1f, onehot).reshape(80, 9 * DP)
    b1r = jnp.repeat(conv1_b, L).reshape(80, 1)
    w2r = conv2_w[0].transpose(2, 3, 0, 1).reshape(9, 80)

    NB = 8
    flat = pl.pallas_call(
        functools.partial(_conv_kernel, nb=NB),
        grid=(B // NB,),
        in_specs=[
            pl.BlockSpec((NB, DP, 2 * PAD + HW), lambda i: (i, 0, 0)),
            pl.BlockSpec((80, 9 * DP), lambda i: (0, 0)),
            pl.BlockSpec((80, 1), lambda i: (0, 0)),
            pl.BlockSpec((9, 80), lambda i: (0, 0)),
            pl.BlockSpec((1, 1), lambda i: (0, 0)),
        ],
        out_specs=pl.BlockSpec((NB, HW), lambda i: (i, 0)),
        out_shape=jax.ShapeDtypeStruct((B, HW), f32),
    )(xp, m1, b1r, w2r, conv2_b.reshape(1, 1))

    noise_t = jax.random.normal(jax.random.key(42), (B, E), f32)
    noise_a = jax.random.normal(jax.random.key(43), (B, E), f32)
    wg = jnp.stack([t_w_gate, a_w_gate])
    wn = jnp.stack([t_w_noise, a_w_noise])
    nz = jnp.stack([noise_t, noise_a])

    gates, aux = pl.pallas_call(
        _gate_kernel,
        in_specs=[
            pl.BlockSpec((B, D), lambda: (0, 0)),
            pl.BlockSpec((2, D, E), lambda: (0, 0, 0)),
            pl.BlockSpec((2, D, E), lambda: (0, 0, 0)),
            pl.BlockSpec((2, B, E), lambda: (0, 0, 0)),
        ],
        out_specs=[
            pl.BlockSpec((2, B, E), lambda: (0, 0, 0)),
            pl.BlockSpec((1, 1), lambda: (0, 0)),
        ],
        out_shape=[
            jax.ShapeDtypeStruct((2, B, E), f32),
            jax.ShapeDtypeStruct((1, 1), f32),
        ],
    )(flat, wg, wn, nz)

    gates_t = gates.transpose(0, 2, 1).reshape(2, E, 1, B)
    HH = H // 2

    yt, ya = pl.pallas_call(
        _moe_kernel,
        grid=(E, 2),
        in_specs=[
            pl.BlockSpec((B, D), lambda e, p: (0, 0)),
            pl.BlockSpec((1, 1, 1, B), lambda e, p: (0, e, 0, 0)),
            pl.BlockSpec((1, 1, 1, B), lambda e, p: (1, e, 0, 0)),
            pl.BlockSpec((1, HH, D), lambda e, p: (e, p, 0)),
            pl.BlockSpec((1, 1, 1, HH), lambda e, p: (e, p, 0, 0)),
            pl.BlockSpec((1, D, HH), lambda e, p: (e, 0, p)),
            pl.BlockSpec((1, 1, D), lambda e, p: (e, 0, 0)),
            pl.BlockSpec((1, HH, D), lambda e, p: (e, p, 0)),
            pl.BlockSpec((1, 1, 1, HH), lambda e, p: (e, p, 0, 0)),
            pl.BlockSpec((1, D, HH), lambda e, p: (e, 0, p)),
            pl.BlockSpec((1, 1, D), lambda e, p: (e, 0, 0)),
        ],
        out_specs=[
            pl.BlockSpec((B, D), lambda e, p: (0, 0)),
            pl.BlockSpec((B, D), lambda e, p: (0, 0)),
        ],
        out_shape=[
            jax.ShapeDtypeStruct((B, D), f32),
            jax.ShapeDtypeStruct((B, D), f32),
        ],
        scratch_shapes=[
            pltpu.VMEM((B, D), f32),
            pltpu.VMEM((B, D), f32),
        ],
        compiler_params=pltpu.CompilerParams(
            dimension_semantics=("arbitrary", "arbitrary")),
    )(flat, gates_t, gates_t,
      t_fc1w, t_fc1b.reshape(E, 2, 1, HH), t_fc2w, t_fc2b.reshape(E, 1, D),
      a_fc1w, a_fc1b.reshape(E, 2, 1, HH), a_fc2w, a_fc2b.reshape(E, 1, D))

    NB2 = 16
    out = pl.pallas_call(
        _combine_kernel,
        grid=(B // NB2,),
        in_specs=[
            pl.BlockSpec((NB2, L, HW), lambda i: (i, 0, 0)),
            pl.BlockSpec((NB2, HW), lambda i: (i, 0)),
            pl.BlockSpec((NB2, HW), lambda i: (i, 0)),
        ],
        out_specs=pl.BlockSpec((NB2, L, HW), lambda i: (i, 0, 0)),
        out_shape=jax.ShapeDtypeStruct((B, L, HW), f32),
    )(x0, yt, ya)

    return out.reshape(B, L, 1, W, W), aux[0, 0]
